# baseline (speedup 1.0000x reference)
import functools

import jax
import jax.numpy as jnp
from jax import lax
from jax.experimental import pallas as pl
from jax.experimental.pallas import tpu as pltpu
from jax.experimental.pallas import tpu_sc as plsc

V = 256
B = 65536
P = B
N_SUM = B - 1
NC = 2
NS = 16
NW = NC * NS
C = P // NW
L = 16
CHUNK = 128


def _gather_body(bat_hbm, uni_hbm, big_hbm, tri_hbm,
                 g1_hbm, g2_hbm, g3_hbm, ps_hbm, hist_hbm,
                 bvec, univ, histv, prow, tcol, bi_idx,
                 g1b, g2b, g3b, row0, row1, psb, sem, sem2):
    wid = lax.axis_index("s") * NC + lax.axis_index("c")
    base = wid * C

    zero0 = jnp.zeros((L,), jnp.float32)
    bvec[pl.ds(C, L)] = jnp.zeros((L,), jnp.int32)

    @pl.when(wid < NW - 1)
    def _():
        pltpu.sync_copy(bat_hbm.at[pl.ds(base, C + 8)], bvec.at[pl.ds(0, C + 8)])

    @pl.when(wid == NW - 1)
    def _():
        pltpu.sync_copy(bat_hbm.at[pl.ds(base, C)], bvec.at[pl.ds(0, C)])

    pltpu.sync_copy(uni_hbm, univ)

    lane = lax.iota(jnp.int32, L)

    for t in range(V // L):
        histv[pl.ds(t * L, L)] = zero0

    def step(j, _):
        off = j * L
        b0 = bvec[pl.ds(off, L)]
        b1 = plsc.load_gather(bvec, [lane + (off + 1)])
        b2 = plsc.load_gather(bvec, [lane + (off + 2)])
        prow[pl.ds(off, L)] = b0 * V + b1
        tcol[pl.ds(off, L)] = b2
        bi_idx[pl.ds(off, L)] = b1 * V + b2
        g1b[pl.ds(off, L)] = plsc.load_gather(univ, [b2])
        m = (base + off + lane) < N_SUM
        plsc.addupdate_scatter(histv, [b1], jnp.where(m, 1.0, 0.0))
        return 0

    lax.fori_loop(0, C // L, step, 0)

    el_copies = []
    for k in range(C // CHUNK):
        sl = pl.ds(k * CHUNK, CHUNK)
        el_copies.append(pltpu.async_copy(big_hbm.at[bi_idx.at[sl]],
                                          g2b.at[sl], sem2))

    tri2d = tri_hbm.reshape(V * V, V)
    rows = (row0, row1)
    nchunk = C // CHUNK

    def fire(k):
        return pltpu.async_copy(tri2d.at[prow.at[pl.ds(k * CHUNK, CHUNK)]],
                                rows[k % 2], sem)

    ps2 = zero0
    fire(0)
    for k in range(nchunk):
        if k + 1 < nchunk:
            fire(k + 1)
        pltpu.make_async_copy(tri2d.at[prow.at[pl.ds(k * CHUNK, CHUNK)]],
                              rows[k % 2], sem).wait()
        rbuf = rows[k % 2]
        for j in range(CHUNK // L):
            off = k * CHUNK + j * L
            cols = tcol[pl.ds(off, L)]
            g3b[pl.ds(off, L)] = plsc.load_gather(rbuf, [lane + j * L, cols])

        def rowacc(i, acc):
            rsum = rbuf[i, pl.ds(0, L)]
            for t in range(1, V // L):
                rsum = rsum + rbuf[i, pl.ds(t * L, L)]
            m = (base + k * CHUNK + i + 0 * lane) < N_SUM
            return acc + jnp.where(m, rsum, 0.0)

        ps2 = lax.fori_loop(0, CHUNK, rowacc, ps2)

    for cp in el_copies:
        cp.wait()

    for t in range(128 // L):
        psb[pl.ds(t * L, L)] = zero0
    psb[pl.ds(0, L)] = ps2
    out_sl = pl.ds(base, C)
    pltpu.sync_copy(g1b, g1_hbm.at[out_sl])
    pltpu.sync_copy(g2b, g2_hbm.at[out_sl])
    pltpu.sync_copy(g3b, g3_hbm.at[out_sl])
    pltpu.sync_copy(psb, ps_hbm.at[pl.ds(wid * 128, 128)])
    pltpu.sync_copy(histv, hist_hbm.at[pl.ds(wid * V, V)])


def _sc_gather(bpad, unigrams, big_flat, tri_flat):
    mesh = plsc.VectorSubcoreMesh(core_axis_name="c", subcore_axis_name="s",
                                  num_cores=NC, num_subcores=NS)
    f32 = jnp.float32
    call = pl.kernel(
        _gather_body,
        out_type=[
            jax.ShapeDtypeStruct((P,), f32),
            jax.ShapeDtypeStruct((P,), f32),
            jax.ShapeDtypeStruct((P,), f32),
            jax.ShapeDtypeStruct((NW * 128,), f32),
            jax.ShapeDtypeStruct((NW * V,), f32),
        ],
        mesh=mesh,
        compiler_params=pltpu.CompilerParams(needs_layout_passes=False,
                                             use_tc_tiling_on_sc=True),
        scratch_types=[
            pltpu.VMEM((C + L,), jnp.int32),
            pltpu.VMEM((V,), f32),
            pltpu.VMEM((V,), f32),
            pltpu.VMEM((C,), jnp.int32),
            pltpu.VMEM((C,), jnp.int32),
            pltpu.VMEM((C,), jnp.int32),
            pltpu.VMEM((C,), f32),
            pltpu.VMEM((C,), f32),
            pltpu.VMEM((C,), f32),
            pltpu.VMEM((CHUNK, V), f32),
            pltpu.VMEM((CHUNK, V), f32),
            pltpu.VMEM((128,), f32),
            pltpu.SemaphoreType.DMA,
            pltpu.SemaphoreType.DMA,
        ],
    )
    return call(bpad, unigrams, big_flat, tri_flat)


def _combine_body(w_ref, ps_ref, hist_ref, big_ref, uni_ref,
                  g1_ref, g2_ref, g3_ref, o_ref):
    usum = jnp.sum(uni_ref[...])
    s2 = jnp.sum(ps_ref[...][:, 0:L])
    rowsum1 = jnp.sum(big_ref[...], axis=1)
    histg = jnp.sum(hist_ref[...], axis=0)
    s1 = jnp.sum(histg * rowsum1)
    w0 = w_ref[0, 0]
    w1 = w_ref[0, 1]
    w2 = w_ref[0, 2]
    w3 = w_ref[0, 3]

    def rb(x):
        return x.astype(jnp.bfloat16).astype(jnp.float32)

    p1 = rb(g1_ref[...] / usum)
    p2 = rb(g2_ref[...] / s1)
    p3 = rb(g3_ref[...] / s2)
    full = (w0 * (1.0 / V)) + p1 * w1 + p2 * w2 + p3 * w3
    o_ref[...] = lax.slice(full, (0,), (B - 2,))


def _combine(w, ps, hist, bigrams, uni, g1, g2, g3):
    return pl.pallas_call(
        _combine_body,
        in_specs=[
            pl.BlockSpec(memory_space=pltpu.SMEM),
            pl.BlockSpec((NW, 128), lambda: (0, 0)),
            pl.BlockSpec((NW, V), lambda: (0, 0)),
            pl.BlockSpec((V, V), lambda: (0, 0)),
            pl.BlockSpec((V,), lambda: (0,)),
            pl.BlockSpec((P,), lambda: (0,)),
            pl.BlockSpec((P,), lambda: (0,)),
            pl.BlockSpec((P,), lambda: (0,)),
        ],
        out_specs=pl.BlockSpec((B - 2,), lambda: (0,)),
        out_shape=jax.ShapeDtypeStruct((B - 2,), jnp.float32),
    )(w, ps, hist, bigrams, uni, g1, g2, g3)


def kernel(batch, unigrams, bigrams, trigrams_tab, w):
    g1, g2, g3, psf, histf = _sc_gather(batch, unigrams,
                                        bigrams.reshape(V * V), trigrams_tab)
    return _combine(
        w.astype(jnp.bfloat16).astype(jnp.float32),
        psf.reshape(NW, 128), histf.reshape(NW, V), bigrams,
        unigrams, g1, g2, g3,
    )


# speedup vs baseline: 1.0287x; 1.0287x over previous
import functools

import jax
import jax.numpy as jnp
from jax import lax
from jax.experimental import pallas as pl
from jax.experimental.pallas import tpu as pltpu
from jax.experimental.pallas import tpu_sc as plsc

V = 256
B = 65536
P = B
N_SUM = B - 1
NC = 2
NS = 16
NW = NC * NS
C = P // NW
L = 16
CHUNK = 128


def _rowsum_body(big_ref, rs1_ref):
    rs1_ref[...] = jnp.sum(big_ref[...], axis=1, keepdims=True)


def _rowsums(bigrams):
    return pl.pallas_call(
        _rowsum_body,
        in_specs=[pl.BlockSpec((V, V), lambda: (0, 0))],
        out_specs=pl.BlockSpec((V, 1), lambda: (0, 0)),
        out_shape=jax.ShapeDtypeStruct((V, 1), jnp.float32),
    )(bigrams)


def _gather_body(bat_hbm, uni_hbm, big_hbm, tri_hbm, rs1_hbm,
                 g1_hbm, g2_hbm, g3_hbm, ps_hbm,
                 bvec, univ, rs1v, prow, tcol, bi_idx,
                 g1b, g2b, g3b, row0, row1, psb, sem, sem2):
    wid = lax.axis_index("s") * NC + lax.axis_index("c")
    base = wid * C

    zero0 = jnp.zeros((L,), jnp.float32)
    bvec[pl.ds(C, L)] = jnp.zeros((L,), jnp.int32)

    @pl.when(wid < NW - 1)
    def _():
        pltpu.sync_copy(bat_hbm.at[pl.ds(base, C + 8)], bvec.at[pl.ds(0, C + 8)])

    @pl.when(wid == NW - 1)
    def _():
        pltpu.sync_copy(bat_hbm.at[pl.ds(base, C)], bvec.at[pl.ds(0, C)])

    pltpu.sync_copy(uni_hbm, univ)
    pltpu.sync_copy(rs1_hbm, rs1v)

    lane = lax.iota(jnp.int32, L)

    def step(j, ps1):
        off = j * L
        b0 = bvec[pl.ds(off, L)]
        b1 = plsc.load_gather(bvec, [lane + (off + 1)])
        b2 = plsc.load_gather(bvec, [lane + (off + 2)])
        prow[pl.ds(off, L)] = b0 * V + b1
        tcol[pl.ds(off, L)] = b2
        bi_idx[pl.ds(off, L)] = b1 * V + b2
        g1b[pl.ds(off, L)] = plsc.load_gather(univ, [b2])
        rs1g = plsc.load_gather(rs1v, [b1])
        m = (base + off + lane) < N_SUM
        return ps1 + jnp.where(m, rs1g, 0.0)

    ps1 = lax.fori_loop(0, C // L, step, zero0)

    el_copies = []
    for k in range(C // CHUNK):
        sl = pl.ds(k * CHUNK, CHUNK)
        el_copies.append(pltpu.async_copy(big_hbm.at[bi_idx.at[sl]],
                                          g2b.at[sl], sem2))

    tri2d = tri_hbm.reshape(V * V, V)
    rows = (row0, row1)
    nchunk = C // CHUNK

    def fire(k):
        return pltpu.async_copy(tri2d.at[prow.at[pl.ds(k * CHUNK, CHUNK)]],
                                rows[k % 2], sem)

    ps2 = zero0
    fire(0)
    for k in range(nchunk):
        if k + 1 < nchunk:
            fire(k + 1)
        pltpu.make_async_copy(tri2d.at[prow.at[pl.ds(k * CHUNK, CHUNK)]],
                              rows[k % 2], sem).wait()
        rbuf = rows[k % 2]
        for j in range(CHUNK // L):
            off = k * CHUNK + j * L
            cols = tcol[pl.ds(off, L)]
            g3b[pl.ds(off, L)] = plsc.load_gather(rbuf, [lane + j * L, cols])

        def rowacc(i, acc):
            rsum = rbuf[i, pl.ds(0, L)]
            for t in range(1, V // L):
                rsum = rsum + rbuf[i, pl.ds(t * L, L)]
            m = (base + k * CHUNK + i + 0 * lane) < N_SUM
            return acc + jnp.where(m, rsum, 0.0)

        ps2 = lax.fori_loop(0, CHUNK, rowacc, ps2)

    for cp in el_copies:
        cp.wait()

    for t in range(128 // L):
        psb[pl.ds(t * L, L)] = zero0
    psb[pl.ds(0, L)] = ps1
    psb[pl.ds(L, L)] = ps2
    out_sl = pl.ds(base, C)
    pltpu.sync_copy(g1b, g1_hbm.at[out_sl])
    pltpu.sync_copy(g2b, g2_hbm.at[out_sl])
    pltpu.sync_copy(g3b, g3_hbm.at[out_sl])
    pltpu.sync_copy(psb, ps_hbm.at[pl.ds(wid * 128, 128)])


def _sc_gather(bpad, unigrams, big_flat, tri_flat, rs1):
    mesh = plsc.VectorSubcoreMesh(core_axis_name="c", subcore_axis_name="s",
                                  num_cores=NC, num_subcores=NS)
    f32 = jnp.float32
    call = pl.kernel(
        _gather_body,
        out_type=[
            jax.ShapeDtypeStruct((P,), f32),
            jax.ShapeDtypeStruct((P,), f32),
            jax.ShapeDtypeStruct((P,), f32),
            jax.ShapeDtypeStruct((NW * 128,), f32),
        ],
        mesh=mesh,
        compiler_params=pltpu.CompilerParams(needs_layout_passes=False,
                                             use_tc_tiling_on_sc=True),
        scratch_types=[
            pltpu.VMEM((C + L,), jnp.int32),
            pltpu.VMEM((V,), f32),
            pltpu.VMEM((V,), f32),
            pltpu.VMEM((C,), jnp.int32),
            pltpu.VMEM((C,), jnp.int32),
            pltpu.VMEM((C,), jnp.int32),
            pltpu.VMEM((C,), f32),
            pltpu.VMEM((C,), f32),
            pltpu.VMEM((C,), f32),
            pltpu.VMEM((CHUNK, V), f32),
            pltpu.VMEM((CHUNK, V), f32),
            pltpu.VMEM((128,), f32),
            pltpu.SemaphoreType.DMA,
            pltpu.SemaphoreType.DMA,
        ],
    )
    return call(bpad, unigrams, big_flat, tri_flat, rs1)


def _combine_body(w_ref, ps_ref, uni_ref, g1_ref, g2_ref, g3_ref, o_ref):
    usum = jnp.sum(uni_ref[...])
    ps = ps_ref[...]
    s1 = jnp.sum(ps[:, 0:L])
    s2 = jnp.sum(ps[:, L:2 * L])
    w0 = w_ref[0, 0]
    w1 = w_ref[0, 1]
    w2 = w_ref[0, 2]
    w3 = w_ref[0, 3]

    def rb(x):
        return x.astype(jnp.bfloat16).astype(jnp.float32)

    p1 = rb(g1_ref[...] / usum)
    p2 = rb(g2_ref[...] / s1)
    p3 = rb(g3_ref[...] / s2)
    full = (w0 * (1.0 / V)) + p1 * w1 + p2 * w2 + p3 * w3
    o_ref[...] = lax.slice(full, (0,), (B - 2,))


def _combine(w, ps, uni, g1, g2, g3):
    return pl.pallas_call(
        _combine_body,
        in_specs=[
            pl.BlockSpec(memory_space=pltpu.SMEM),
            pl.BlockSpec((NW, 128), lambda: (0, 0)),
            pl.BlockSpec((V,), lambda: (0,)),
            pl.BlockSpec((P,), lambda: (0,)),
            pl.BlockSpec((P,), lambda: (0,)),
            pl.BlockSpec((P,), lambda: (0,)),
        ],
        out_specs=pl.BlockSpec((B - 2,), lambda: (0,)),
        out_shape=jax.ShapeDtypeStruct((B - 2,), jnp.float32),
    )(w, ps, uni, g1, g2, g3)


def kernel(batch, unigrams, bigrams, trigrams_tab, w):
    rs1 = _rowsums(bigrams)
    g1, g2, g3, psf = _sc_gather(batch, unigrams, bigrams.reshape(V * V),
                                 trigrams_tab, rs1.reshape(V))
    return _combine(
        w.astype(jnp.bfloat16).astype(jnp.float32),
        psf.reshape(NW, 128), unigrams, g1, g2, g3,
    )
